# Initial kernel scaffold; baseline (speedup 1.0000x reference)
import jax
import jax.numpy as jnp
from jax.experimental import pallas as pl


def kernel(obs, W1, b1, W2, b2, Wout, bout, Wmu, bmu, log_std, V1, Vb1, V2, Vb2, V3, Vb3, centers, stats_mean, stats_var):
    raise NotImplementedError("write your pallas kernel here")


# speedup vs baseline: 2.5423x; 2.5423x over previous
import functools

import jax
import jax.numpy as jnp
from jax.experimental import pallas as pl

B = 8192
OBS = 1024
ACT = 32
M = 16
HID = 256
EPS_W = 1e-06

BM = 512


def _fused_kernel(obs_ref, w1_ref, b1_ref, w2_ref, b2_ref, wmu_ref, bmu_ref,
                  lsd_ref, v1_ref, vb1_ref, v2_ref, vb2_ref, v3_ref, vb3_ref,
                  cen_ref, sm_ref, sv_ref,
                  mu_ref, ls_ref, val_ref, idx_ref):
    obs = obs_ref[...]
    h1 = jnp.tanh(jnp.dot(obs, w1_ref[...],
                          preferred_element_type=jnp.float32) + b1_ref[...])
    feat = jnp.tanh(jnp.dot(h1, w2_ref[...],
                            preferred_element_type=jnp.float32) + b2_ref[...])

    z = (feat - sm_ref[...]) / jnp.sqrt(sv_ref[...] + EPS_W)

    best_d = jnp.full((BM, 1), jnp.inf, dtype=jnp.float32)
    best_i = jnp.zeros((BM, 1), dtype=jnp.int32)
    for m in range(M):
        diff = z - cen_ref[m, :]
        d = jnp.sum(diff * diff, axis=1, keepdims=True)
        upd = d < best_d
        best_d = jnp.where(upd, d, best_d)
        best_i = jnp.where(upd, m, best_i)

    mu_all = jnp.dot(feat, wmu_ref[...],
                     preferred_element_type=jnp.float32) + bmu_ref[...]
    mu = jnp.zeros((BM, ACT), dtype=jnp.float32)
    ls = jnp.zeros((BM, ACT), dtype=jnp.float32)
    for m in range(M):
        sel = (best_i == m).astype(jnp.float32)
        mu = mu + sel * mu_all[:, m * ACT:(m + 1) * ACT]
        ls = ls + sel * lsd_ref[m, :]

    v = jnp.tanh(jnp.dot(feat, v1_ref[...],
                         preferred_element_type=jnp.float32) + vb1_ref[...])
    v = jnp.tanh(jnp.dot(v, v2_ref[...],
                         preferred_element_type=jnp.float32) + vb2_ref[...])
    val = jnp.sum(v * v3_ref[...], axis=1, keepdims=True) + vb3_ref[...]

    mu_ref[...] = mu
    ls_ref[...] = ls
    val_ref[...] = val
    idx_ref[...] = best_i


@jax.jit
def kernel(obs, W1, b1, W2, b2, Wout, bout, Wmu, bmu, log_std,
           V1, Vb1, V2, Vb2, V3, Vb3, centers, stats_mean, stats_var):
    del Wout, bout
    W1t = W1.T
    W2t = W2.T
    Wmut = Wmu.reshape(M * ACT, HID).T
    bmu_r = bmu.reshape(1, M * ACT)
    V1t = V1.T
    V2t = V2.T
    v3_row = V3.reshape(1, HID)
    vb3_row = Vb3.reshape(1, 1)

    grid = (B // BM,)
    row_spec = pl.BlockSpec((BM, OBS), lambda i: (i, 0))
    full = lambda shape: pl.BlockSpec(shape, lambda i: (0,) * len(shape))

    mu, ls, val, idx = pl.pallas_call(
        _fused_kernel,
        grid=grid,
        in_specs=[
            row_spec,
            full((OBS, HID)), full((HID,)),
            full((HID, HID)), full((HID,)),
            full((HID, M * ACT)), full((1, M * ACT)),
            full((M, ACT)),
            full((HID, HID)), full((HID,)),
            full((HID, HID)), full((HID,)),
            full((1, HID)), full((1, 1)),
            full((M, HID)), full((HID,)), full((HID,)),
        ],
        out_specs=[
            pl.BlockSpec((BM, ACT), lambda i: (i, 0)),
            pl.BlockSpec((BM, ACT), lambda i: (i, 0)),
            pl.BlockSpec((BM, 1), lambda i: (i, 0)),
            pl.BlockSpec((BM, 1), lambda i: (i, 0)),
        ],
        out_shape=[
            jax.ShapeDtypeStruct((B, ACT), jnp.float32),
            jax.ShapeDtypeStruct((B, ACT), jnp.float32),
            jax.ShapeDtypeStruct((B, 1), jnp.float32),
            jax.ShapeDtypeStruct((B, 1), jnp.int32),
        ],
    )(obs, W1t, b1, W2t, b2, Wmut, bmu_r, log_std,
      V1t, Vb1, V2t, Vb2, v3_row, vb3_row, centers, stats_mean, stats_var)

    return (mu, ls, val[:, 0], idx[:, 0])
